# baseline (speedup 1.0000x reference)
import jax
import jax.numpy as jnp
from jax.experimental import pallas as pl

_G = 8


def _edgeconv_block(x, Wa, ba, g, bt, Wb, bb, din):
    G, n, _ = x.shape
    dh = Wa.shape[1]
    Wtop = Wa[:din, :]
    Wbot = Wa[din:, :]
    xf = x.reshape(G * n, din)
    a = jnp.dot(xf, Wtop - Wbot, preferred_element_type=jnp.float32, precision=jax.lax.Precision.HIGHEST) + ba
    c = jnp.dot(xf, Wbot, preferred_element_type=jnp.float32, precision=jax.lax.Precision.HIGHEST)
    h = a.reshape(G, n, 1, dh) + c.reshape(G, 1, n, dh)
    mu = jnp.mean(h, axis=-1, keepdims=True)
    var = jnp.mean(jnp.square(h - mu), axis=-1, keepdims=True)
    h = (h - mu) * jax.lax.rsqrt(var + 1e-5) * g + bt
    h = jnp.maximum(h, 0.0)
    hbar = jnp.mean(h, axis=2)
    m = jnp.dot(hbar.reshape(G * n, dh), Wb, preferred_element_type=jnp.float32, precision=jax.lax.Precision.HIGHEST) + bb
    return m.reshape(G, n, -1)


def _critic_kernel(x_ref, W1a_ref, b1a_ref, g1_ref, beta1_ref, W1b_ref, b1b_ref,
                   W2a_ref, b2a_ref, g2_ref, beta2_ref, W2b_ref, b2b_ref,
                   W3a_ref, b3a_ref, g3_ref, beta3_ref, W3b_ref, b3b_ref,
                   Wp1_ref, bp1_ref, Wp2_ref, bp2_ref, Wq1_ref, bq1_ref,
                   Wq2_ref, bq2_ref, out_ref):
    x = x_ref[...]
    G, n, _ = x.shape

    x = jnp.maximum(_edgeconv_block(x, W1a_ref[...], b1a_ref[...], g1_ref[...],
                                    beta1_ref[...], W1b_ref[...], b1b_ref[...], 96), 0.0)
    x = jnp.maximum(_edgeconv_block(x, W2a_ref[...], b2a_ref[...], g2_ref[...],
                                    beta2_ref[...], W2b_ref[...], b2b_ref[...], 32), 0.0)
    x = _edgeconv_block(x, W3a_ref[...], b3a_ref[...], g3_ref[...],
                        beta3_ref[...], W3b_ref[...], b3b_ref[...], 32)

    xf = x.reshape(G * n, 32)
    h = jnp.maximum(jnp.dot(xf, Wp1_ref[...], preferred_element_type=jnp.float32, precision=jax.lax.Precision.HIGHEST) + bp1_ref[...], 0.0)
    h = jnp.maximum(jnp.dot(h, Wp2_ref[...], preferred_element_type=jnp.float32, precision=jax.lax.Precision.HIGHEST) + bp2_ref[...], 0.0)
    pooled = jnp.mean(h.reshape(G, n, 16), axis=1)
    q = jnp.maximum(jnp.dot(pooled, Wq1_ref[...], preferred_element_type=jnp.float32, precision=jax.lax.Precision.HIGHEST) + bq1_ref[...], 0.0)
    y = jnp.dot(q, Wq2_ref[...], preferred_element_type=jnp.float32, precision=jax.lax.Precision.HIGHEST) + bq2_ref[...]
    out_ref[...] = y


def kernel(inputs, W1a, b1a, g1, beta1, W1b, b1b, W2a, b2a, g2, beta2, W2b, b2b,
           W3a, b3a, g3, beta3, W3b, b3b, Wp1, bp1, Wp2, bp2, Wq1, bq1, Wq2, bq2):
    b, t, n, d = inputs.shape
    B = b * t
    x = inputs.reshape(B, n, d)

    row = lambda v: v.reshape(1, -1)
    params = [W1a, row(b1a), row(g1), row(beta1), W1b, row(b1b),
              W2a, row(b2a), row(g2), row(beta2), W2b, row(b2b),
              W3a, row(b3a), row(g3), row(beta3), W3b, row(b3b),
              Wp1, row(bp1), Wp2, row(bp2), Wq1, row(bq1), Wq2, row(bq2)]

    grid = (B // _G,)
    full = lambda p: pl.BlockSpec(p.shape, lambda i: (0,) * p.ndim)
    in_specs = [pl.BlockSpec((_G, n, d), lambda i: (i, 0, 0))] + [full(p) for p in params]
    out_spec = pl.BlockSpec((_G, 1), lambda i: (i, 0))

    y = pl.pallas_call(
        _critic_kernel,
        grid=grid,
        in_specs=in_specs,
        out_specs=out_spec,
        out_shape=jax.ShapeDtypeStruct((B, 1), jnp.float32),
    )(x, *params)
    return y.reshape(b, t, 1)


# speedup vs baseline: 1.5390x; 1.5390x over previous
import jax
import jax.numpy as jnp
from jax.experimental import pallas as pl

_G = 8
_HI = jax.lax.Precision.HIGHEST


def _dot(a, b):
    return jnp.dot(a, b, preferred_element_type=jnp.float32, precision=_HI)


def _edgeconv_block(x, Wa, ba, g, bt, Wb, bb, E, din, dh):
    G, n, _ = x.shape
    p = 128 // dh
    Q = (n * dh) // 128

    Wtop = Wa[:din, :]
    Wbot = Wa[din:, :]
    xf = x.reshape(G * n, din)
    a = _dot(xf, Wtop - Wbot) + ba
    c = _dot(xf, Wbot)

    a_hat = a - jnp.mean(a, axis=-1, keepdims=True)
    c_hat = c - jnp.mean(c, axis=-1, keepdims=True)
    va = jnp.mean(jnp.square(a_hat), axis=-1, keepdims=True)
    vc = jnp.mean(jnp.square(c_hat), axis=-1, keepdims=True)

    s = jnp.sqrt(2.0 / dh)
    ones = jnp.ones((G * n, 1), jnp.float32)
    a_aug = jnp.concatenate([a_hat * s, va, ones], axis=-1).reshape(G, n, dh + 2)
    c_aug = jnp.concatenate([c_hat * s, ones, vc], axis=-1).reshape(G, n, dh + 2)
    var_T = jax.lax.dot_general(
        c_aug, a_aug, (((2,), (2,)), ((0,), (0,))),
        preferred_element_type=jnp.float32, precision=_HI)
    r_T = jax.lax.rsqrt(var_T + 1e-5)

    r_flat = r_T.reshape(G * n, n)
    r_hi = r_flat.astype(jnp.bfloat16)
    r_lo = (r_flat - r_hi.astype(jnp.float32)).astype(jnp.bfloat16)
    r2 = (jnp.dot(r_hi, E, preferred_element_type=jnp.float32)
          + jnp.dot(r_lo, E, preferred_element_type=jnp.float32))
    r2 = r2.reshape(G, n, Q, 128)

    Ag = (g * a_hat).reshape(G, n, dh)
    A2 = jnp.concatenate(
        [Ag[:, u * Q:(u + 1) * Q, :] for u in range(p)], axis=-1)
    Cg = (g * c_hat).reshape(G, n, dh)
    C2 = jnp.concatenate([Cg] * p, axis=-1)
    bt2 = jnp.concatenate([bt] * p, axis=-1)

    t = r2 * (A2[:, None, :, :] + C2[:, :, None, :]) + bt2
    t = jnp.maximum(t, 0.0)
    sj = jnp.sum(t, axis=1)
    hbar = jnp.concatenate(
        [sj[:, :, u * dh:(u + 1) * dh] for u in range(p)], axis=1) * (1.0 / n)
    m = _dot(hbar.reshape(G * n, dh), Wb) + bb
    return m.reshape(G, n, -1)


def _critic_kernel(x_ref, E1_ref, E23_ref,
                   W1a_ref, b1a_ref, g1_ref, beta1_ref, W1b_ref, b1b_ref,
                   W2a_ref, b2a_ref, g2_ref, beta2_ref, W2b_ref, b2b_ref,
                   W3a_ref, b3a_ref, g3_ref, beta3_ref, W3b_ref, b3b_ref,
                   Wp1_ref, bp1_ref, Wp2_ref, bp2_ref, Wq1_ref, bq1_ref,
                   Wq2_ref, bq2_ref, out_ref):
    x = x_ref[...]
    G, n, _ = x.shape
    E1 = E1_ref[...]
    E23 = E23_ref[...]

    x = jnp.maximum(_edgeconv_block(x, W1a_ref[...], b1a_ref[...], g1_ref[...],
                                    beta1_ref[...], W1b_ref[...], b1b_ref[...],
                                    E1, 96, 64), 0.0)
    x = jnp.maximum(_edgeconv_block(x, W2a_ref[...], b2a_ref[...], g2_ref[...],
                                    beta2_ref[...], W2b_ref[...], b2b_ref[...],
                                    E23, 32, 32), 0.0)
    x = _edgeconv_block(x, W3a_ref[...], b3a_ref[...], g3_ref[...],
                        beta3_ref[...], W3b_ref[...], b3b_ref[...], E23, 32, 32)

    xf = x.reshape(G * n, 32)
    h = jnp.maximum(_dot(xf, Wp1_ref[...]) + bp1_ref[...], 0.0)
    h = jnp.maximum(_dot(h, Wp2_ref[...]) + bp2_ref[...], 0.0)
    pooled = jnp.mean(h.reshape(G, n, 16), axis=1)
    q = jnp.maximum(_dot(pooled, Wq1_ref[...]) + bq1_ref[...], 0.0)
    y = _dot(q, Wq2_ref[...]) + bq2_ref[...]
    out_ref[...] = y


def _selection_matrix(n, dh):
    p = 128 // dh
    Q = (n * dh) // 128
    cols = jnp.arange(Q * 128)
    target = (cols % 128) // dh * Q + cols // 128
    rows = jnp.arange(n)
    return (rows[:, None] == target[None, :]).astype(jnp.bfloat16)


def kernel(inputs, W1a, b1a, g1, beta1, W1b, b1b, W2a, b2a, g2, beta2, W2b, b2b,
           W3a, b3a, g3, beta3, W3b, b3b, Wp1, bp1, Wp2, bp2, Wq1, bq1, Wq2, bq2):
    b, t, n, d = inputs.shape
    B = b * t
    x = inputs.reshape(B, n, d)

    E1 = _selection_matrix(n, 64)
    E23 = _selection_matrix(n, 32)

    row = lambda v: v.reshape(1, -1)
    params = [E1, E23,
              W1a, row(b1a), row(g1), row(beta1), W1b, row(b1b),
              W2a, row(b2a), row(g2), row(beta2), W2b, row(b2b),
              W3a, row(b3a), row(g3), row(beta3), W3b, row(b3b),
              Wp1, row(bp1), Wp2, row(bp2), Wq1, row(bq1), Wq2, row(bq2)]

    grid = (B // _G,)
    full = lambda p: pl.BlockSpec(p.shape, lambda i: (0,) * p.ndim)
    in_specs = [pl.BlockSpec((_G, n, d), lambda i: (i, 0, 0))] + [full(p) for p in params]
    out_spec = pl.BlockSpec((_G, 1), lambda i: (i, 0))

    y = pl.pallas_call(
        _critic_kernel,
        grid=grid,
        in_specs=in_specs,
        out_specs=out_spec,
        out_shape=jax.ShapeDtypeStruct((B, 1), jnp.float32),
    )(x, *params)
    return y.reshape(b, t, 1)
